# baseline (speedup 1.0000x reference)
import functools

import jax
import jax.numpy as jnp
from jax import lax
from jax.experimental import pallas as pl
from jax.experimental.pallas import tpu as pltpu
from jax.experimental.pallas import tpu_sc as plsc

B = 16384
D = 64
A = 16
V = 1_000_000
M0 = 10000
M1 = 100

_info = plsc.get_sparse_core_info()
NC, NS, L = _info.num_cores, _info.num_subcores, _info.num_lanes
NW = NC * NS
BPW = B // NW
CHUNK = 128
NCHUNK = BPW // CHUNK
GPC = CHUNK // L

BLK = 2048
NBLK = -(-V // BLK)
P2_ROWS = NBLK * BLK // 8


def _proj_body(wt_ref, tt_ref, b_ref, out_ref):
    chunk = (
        jax.lax.dot_general(
            wt_ref[...], tt_ref[...], (((1,), (0,)), ((), ())),
            preferred_element_type=jnp.float32,
        )
        + b_ref[...]
    )
    p2 = jnp.transpose(chunk.reshape(A, BLK // 8, 8), (1, 2, 0))
    out_ref[...] = p2.reshape(BLK // 8, 8 * A)


def _tc_project(table_t, w_t, b2):
    return pl.pallas_call(
        _proj_body,
        grid=(NBLK,),
        in_specs=[
            pl.BlockSpec((A, D), lambda i: (0, 0)),
            pl.BlockSpec((D, BLK), lambda i: (0, i)),
            pl.BlockSpec((A, 1), lambda i: (0, 0)),
        ],
        out_specs=pl.BlockSpec((BLK // 8, 8 * A), lambda i: (i, 0)),
        out_shape=jax.ShapeDtypeStruct((P2_ROWS, 8 * A), jnp.float32),
    )(w_t, table_t, b2)


def _sc_gather(state_flat, p2):
    mesh = plsc.VectorSubcoreMesh(core_axis_name="c", subcore_axis_name="s")

    @functools.partial(
        pl.kernel,
        mesh=mesh,
        compiler_params=pltpu.CompilerParams(
            needs_layout_passes=False, use_tc_tiling_on_sc=True
        ),
        out_type=jax.ShapeDtypeStruct((A * B,), jnp.float32),
        scratch_types=[
            pltpu.VMEM((BPW * 3,), jnp.int32),
            pltpu.VMEM((BPW,), jnp.int32),
            pltpu.VMEM((NCHUNK, CHUNK), jnp.int32),
            pltpu.VMEM((BPW, 8 * A), jnp.float32),
            pltpu.VMEM((A * BPW,), jnp.float32),
            pltpu.SemaphoreType.DMA,
        ],
    )
    def k(state_hbm, p2_hbm, out_hbm,
          state_v, ids_v, q_v, rows_v, out_v, sem):
        wid = lax.axis_index("s") * NC + lax.axis_index("c")
        base = wid * BPW
        pltpu.sync_copy(state_hbm.at[pl.ds(base * 3, BPW * 3)], state_v)
        lanes3 = lax.iota(jnp.int32, L) * 3
        for g in range(BPW // L):
            r3 = lanes3 + g * (L * 3)
            s0 = plsc.load_gather(state_v, [r3])
            s1 = plsc.load_gather(state_v, [r3 + 1])
            s2 = plsc.load_gather(state_v, [r3 + 2])
            ids = s0 * M0 + s1 * M1 + s2
            ids_v[pl.ds(g * L, L)] = ids
            q_v[g // GPC, pl.ds((g % GPC) * L, L)] = ids >> 3
        copies = [
            pltpu.async_copy(
                p2_hbm.at[q_v.at[cg]],
                rows_v.at[pl.ds(cg * CHUNK, CHUNK)],
                sem,
            )
            for cg in range(NCHUNK)
        ]
        for c in copies:
            c.wait()
        lanes = lax.iota(jnp.int32, L)
        for g in range(BPW // L):
            ids = ids_v[pl.ds(g * L, L)]
            row_idx = lanes + g * L
            col_base = (ids & 7) * A
            for j in range(A):
                vals = plsc.load_gather(rows_v, [row_idx, col_base + j])
                plsc.store_scatter(out_v, [row_idx + j * BPW], vals)
        for j in range(A):
            pltpu.sync_copy(
                out_v.at[pl.ds(j * BPW, BPW)],
                out_hbm.at[pl.ds(j * B + base, BPW)],
            )

    return k(state_flat, p2)


def kernel(state, table, W, b):
    p2 = _tc_project(table.T, W.T, b.reshape(A, 1))
    flat = _sc_gather(state.reshape(-1), p2)
    return flat.reshape(A, B).T


# speedup vs baseline: 1.5273x; 1.5273x over previous
import functools

import jax
import jax.numpy as jnp
from jax import lax
from jax.experimental import pallas as pl
from jax.experimental.pallas import tpu as pltpu
from jax.experimental.pallas import tpu_sc as plsc

B = 16384
D = 64
A = 16
V = 1_000_000
M0 = 10000
M1 = 100

_info = plsc.get_sparse_core_info()
NC, NS = _info.num_cores, _info.num_subcores
NW = NC * NS
BPW = B // NW
CI = 128
NCHUNK = BPW // CI

BLKB = 2048

DP = 2 * D
VP = V // 2


def _ids_body(state_ref, hi_ref, par_ref):
    s = state_ref[...]
    ids = s[:, 0:1] * M0 + s[:, 1:2] * M1 + s[:, 2:3]
    hi_ref[...] = ids >> 1
    par_ref[...] = ids & 1


def _tc_ids(state):
    return pl.pallas_call(
        _ids_body,
        out_shape=(
            jax.ShapeDtypeStruct((B, 1), jnp.int32),
            jax.ShapeDtypeStruct((B, 1), jnp.int32),
        ),
    )(state)


def _sc_gather(ids3, table):
    mesh = plsc.VectorSubcoreMesh(core_axis_name="c", subcore_axis_name="s")

    @functools.partial(
        pl.kernel,
        mesh=mesh,
        out_type=jax.ShapeDtypeStruct((B, DP), jnp.float32),
        scratch_types=[
            pltpu.VMEM((NCHUNK, CI), jnp.int32),
            pltpu.VMEM((BPW, DP), jnp.float32),
            pltpu.SemaphoreType.DMA,
        ],
    )
    def k(ids_hbm, table_hbm, emb_hbm, idx_v, rows_v, sem):
        wid = lax.axis_index("s") * NC + lax.axis_index("c")
        base = wid * BPW
        pltpu.sync_copy(ids_hbm.at[wid], idx_v)
        copies = [
            pltpu.async_copy(
                table_hbm.at[idx_v.at[c]],
                rows_v.at[pl.ds(c * CI, CI)],
                sem,
            )
            for c in range(NCHUNK)
        ]
        for c in copies:
            c.wait()
        pltpu.sync_copy(rows_v, emb_hbm.at[pl.ds(base, BPW)])

    return k(ids3, table)


def _head_body(packed_ref, par_ref, w_ref, b_ref, out_ref):
    p = packed_ref[...]
    emb = jnp.where(par_ref[...] == 0, p[:, :D], p[:, D:])
    out_ref[...] = (
        jax.lax.dot_general(
            emb, w_ref[...], (((1,), (0,)), ((), ())),
            preferred_element_type=jnp.float32,
        )
        + b_ref[...]
    )


def _tc_head(packed, par, w, b2):
    return pl.pallas_call(
        _head_body,
        grid=(B // BLKB,),
        in_specs=[
            pl.BlockSpec((BLKB, DP), lambda i: (i, 0)),
            pl.BlockSpec((BLKB, 1), lambda i: (i, 0)),
            pl.BlockSpec((D, A), lambda i: (0, 0)),
            pl.BlockSpec((1, A), lambda i: (0, 0)),
        ],
        out_specs=pl.BlockSpec((BLKB, A), lambda i: (i, 0)),
        out_shape=jax.ShapeDtypeStruct((B, A), jnp.float32),
    )(packed, par, w, b2)


def kernel(state, table, W, b):
    hi, par = _tc_ids(state)
    packed = _sc_gather(hi.reshape(NW, NCHUNK, CI), table.reshape(VP, DP))
    return _tc_head(packed, par, W, b.reshape(1, A))
